# baseline (speedup 1.0000x reference)
import functools

import jax
import jax.numpy as jnp
from jax import lax
from jax.experimental import pallas as pl
from jax.experimental.pallas import tpu as pltpu
from jax.experimental.pallas import tpu_sc as plsc

N = 10000
E = 320000
D = 128
G = 64

NC = 2
NS = 16
NW = NC * NS
EPT = E // NW
CH = 128
NCH = 79
EPT_PAD = NCH * CH
ZROWS = 632
N_PAD = ZROWS * NS
OROWS = 632
OTAIL = N - 15 * OROWS

BN = 2000
GRID = N // BN


def _sc_aggregate(h, srcp, dstp, zeros):
    mesh = plsc.VectorSubcoreMesh(core_axis_name="c", subcore_axis_name="s")

    @functools.partial(
        pl.kernel,
        out_type=jax.ShapeDtypeStruct((NC, N, D), jnp.float32),
        mesh=mesh,
        scratch_types=[
            pltpu.VMEM((NCH, CH), jnp.int32),
            pltpu.VMEM((NCH, CH), jnp.int32),
            pltpu.VMEM((CH, D), jnp.float32),
            pltpu.VMEM_SHARED((N_PAD, D), jnp.float32),
            pltpu.SemaphoreType.DMA,
        ],
        compiler_params=pltpu.CompilerParams(use_tc_tiling_on_sc=False),
    )
    def agg_kernel(h_hbm, srcp_hbm, dstp_hbm, zeros_hbm, out_hbm,
                   src_v, dst_v, rows_v, acc_sh, sem):
        c = lax.axis_index("c")
        s = lax.axis_index("s")
        t = c * NS + s
        pltpu.sync_copy(zeros_hbm.at[pl.ds(s * ZROWS, ZROWS)],
                        acc_sh.at[pl.ds(s * ZROWS, ZROWS)])
        pltpu.sync_copy(srcp_hbm.at[t], src_v)
        pltpu.sync_copy(dstp_hbm.at[t], dst_v)
        plsc.subcore_barrier()

        def body(j, carry):
            pltpu.async_copy(h_hbm.at[src_v.at[j]], rows_v, sem).wait()
            pltpu.sync_copy(rows_v, acc_sh.at[dst_v.at[j]], add=True)
            return carry

        lax.fori_loop(0, NCH, body, 0)
        plsc.subcore_barrier()

        @pl.when(s < NS - 1)
        def _copy_full():
            pltpu.sync_copy(acc_sh.at[pl.ds(s * OROWS, OROWS)],
                            out_hbm.at[c, pl.ds(s * OROWS, OROWS)])

        @pl.when(s == NS - 1)
        def _copy_tail():
            pltpu.sync_copy(acc_sh.at[pl.ds((NS - 1) * OROWS, OTAIL)],
                            out_hbm.at[c, pl.ds((NS - 1) * OROWS, OTAIL)])

    return agg_kernel(h, srcp, dstp, zeros)


def _leaky(x):
    return jnp.where(x >= 0, x, 0.01 * x)


def _tc_mlp(h, agg, W1, b1, W2, b2, g, be, do_ln):
    def body(x_ref, a_ref, W1_ref, b1_ref, W2_ref, b2_ref, g_ref, be_ref, o_ref):
        t = x_ref[...] + a_ref[0] + a_ref[1]
        u = jnp.dot(t, W1_ref[...], preferred_element_type=jnp.float32) + b1_ref[...]
        u = _leaky(u)
        v = jnp.dot(u, W2_ref[...], preferred_element_type=jnp.float32) + b2_ref[...]
        if do_ln:
            v = _leaky(v)
            m = jnp.mean(v, axis=1, keepdims=True)
            var = jnp.mean((v - m) * (v - m), axis=1, keepdims=True)
            v = (v - m) * lax.rsqrt(var + 1e-5) * g_ref[...] + be_ref[...]
        o_ref[...] = v

    full = lambda i: (0, 0)
    return pl.pallas_call(
        body,
        grid=(GRID,),
        in_specs=[
            pl.BlockSpec((BN, D), lambda i: (i, 0)),
            pl.BlockSpec((NC, BN, D), lambda i: (0, i, 0)),
            pl.BlockSpec((D, D), full),
            pl.BlockSpec((1, D), full),
            pl.BlockSpec((D, D), full),
            pl.BlockSpec((1, D), full),
            pl.BlockSpec((1, D), full),
            pl.BlockSpec((1, D), full),
        ],
        out_specs=pl.BlockSpec((BN, D), lambda i: (i, 0)),
        out_shape=jax.ShapeDtypeStruct((N, D), jnp.float32),
    )(h, agg, W1, b1.reshape(1, D), W2, b2.reshape(1, D),
      g.reshape(1, D), be.reshape(1, D))


def _tc_mlp_pool(h, agg, W1, b1, W2, b2, batch2, Wp1, bp1, Wp2, bp2):
    def body(x_ref, a_ref, W1_ref, b1_ref, W2_ref, b2_ref, b_ref,
             Wp1_ref, bp1_ref, Wp2_ref, bp2_ref, e_ref, o_ref, acc_s, acc_c):
        i = pl.program_id(0)

        @pl.when(i == 0)
        def _init():
            acc_s[...] = jnp.zeros_like(acc_s)
            acc_c[...] = jnp.zeros_like(acc_c)

        t = x_ref[...] + a_ref[0] + a_ref[1]
        u = jnp.dot(t, W1_ref[...], preferred_element_type=jnp.float32) + b1_ref[...]
        u = _leaky(u)
        v = jnp.dot(u, W2_ref[...], preferred_element_type=jnp.float32) + b2_ref[...]
        e_ref[...] = v
        hp = _leaky(v)
        gids = b_ref[...]
        mask = (gids == lax.broadcasted_iota(jnp.int32, (1, G), 1)
                ).astype(jnp.float32)
        dn = (((0,), (0,)), ((), ()))
        acc_s[...] += lax.dot_general(mask, hp, dn,
                                      preferred_element_type=jnp.float32)
        acc_c[...] += lax.dot_general(mask, jnp.ones_like(hp), dn,
                                      preferred_element_type=jnp.float32)

        @pl.when(i == GRID - 1)
        def _fin():
            pooled = acc_s[...] / jnp.maximum(acc_c[...], 1.0)
            o = jnp.dot(pooled, Wp1_ref[...],
                        preferred_element_type=jnp.float32) + bp1_ref[...]
            o = jnp.dot(o, Wp2_ref[...],
                        preferred_element_type=jnp.float32) + bp2_ref[...]
            mx = jnp.max(o, axis=1, keepdims=True)
            lse = jnp.log(jnp.sum(jnp.exp(o - mx), axis=1, keepdims=True))
            o_ref[...] = o - mx - lse

    full = lambda i: (0, 0)
    return pl.pallas_call(
        body,
        grid=(GRID,),
        in_specs=[
            pl.BlockSpec((BN, D), lambda i: (i, 0)),
            pl.BlockSpec((NC, BN, D), lambda i: (0, i, 0)),
            pl.BlockSpec((D, D), full),
            pl.BlockSpec((1, D), full),
            pl.BlockSpec((D, D), full),
            pl.BlockSpec((1, D), full),
            pl.BlockSpec((BN, 1), lambda i: (i, 0)),
            pl.BlockSpec((D, D), full),
            pl.BlockSpec((1, D), full),
            pl.BlockSpec((D, D), full),
            pl.BlockSpec((1, D), full),
        ],
        out_specs=[
            pl.BlockSpec((BN, D), lambda i: (i, 0)),
            pl.BlockSpec((G, D), full),
        ],
        out_shape=[
            jax.ShapeDtypeStruct((N, D), jnp.float32),
            jax.ShapeDtypeStruct((G, D), jnp.float32),
        ],
        scratch_shapes=[
            pltpu.VMEM((G, D), jnp.float32),
            pltpu.VMEM((G, D), jnp.float32),
        ],
    )(h, agg, W1, b1.reshape(1, D), W2, b2.reshape(1, D), batch2,
      Wp1, bp1.reshape(1, D), Wp2, bp2.reshape(1, D))


def kernel(x, edge_index, batch,
           W1_0, b1_0, W2_0, b2_0,
           W1_1, b1_1, W2_1, b2_1,
           W1_2, b1_2, W2_2, b2_2,
           g0, be0, g1, be1,
           Wp1, bp1, Wp2, bp2):
    src = edge_index[0]
    dst = edge_index[1]
    pad = EPT_PAD - EPT
    srcp = jnp.pad(src.reshape(NW, EPT), ((0, 0), (0, pad))
                   ).reshape(NW, NCH, CH)
    dummy = (N + jnp.arange(NW, dtype=jnp.int32) % (N_PAD - N))[:, None]
    dstp = jnp.concatenate(
        [dst.reshape(NW, EPT), jnp.broadcast_to(dummy, (NW, pad))],
        axis=1).reshape(NW, NCH, CH)
    zeros = jnp.zeros((N_PAD, D), jnp.float32)
    batch2 = batch.reshape(N, 1)

    convs = [(W1_0, b1_0, W2_0, b2_0), (W1_1, b1_1, W2_1, b2_1)]
    lns = [(g0, be0), (g1, be1)]

    h = x
    for i in range(2):
        agg = _sc_aggregate(h, srcp, dstp, zeros)
        W1, b1, W2, b2 = convs[i]
        g, be = lns[i]
        h = _tc_mlp(h, agg, W1, b1, W2, b2, g, be, do_ln=True)
    agg = _sc_aggregate(h, srcp, dstp, zeros)
    emb, logp = _tc_mlp_pool(h, agg, W1_2, b1_2, W2_2, b2_2, batch2,
                             Wp1, bp1, Wp2, bp2)
    return (emb, logp)


# speedup vs baseline: 1.0026x; 1.0026x over previous
import functools

import jax
import jax.numpy as jnp
from jax import lax
from jax.experimental import pallas as pl
from jax.experimental.pallas import tpu as pltpu
from jax.experimental.pallas import tpu_sc as plsc

N = 10000
E = 320000
D = 128
G = 64

NC = 2
NS = 16
NW = NC * NS
EPT = E // NW
CH = 128
NCH = 79
EPT_PAD = NCH * CH
ZROWS = 632
N_PAD = ZROWS * NS
OROWS = 632
OTAIL = N - 15 * OROWS

BN = 2000
GRID = N // BN


def _sc_aggregate(h, srcp, dstp, zeros):
    mesh = plsc.VectorSubcoreMesh(core_axis_name="c", subcore_axis_name="s")

    @functools.partial(
        pl.kernel,
        out_type=jax.ShapeDtypeStruct((NC, N, D), jnp.float32),
        mesh=mesh,
        scratch_types=[
            pltpu.VMEM((NCH, CH), jnp.int32),
            pltpu.VMEM((NCH, CH), jnp.int32),
            pltpu.VMEM((CH, D), jnp.float32),
            pltpu.VMEM_SHARED((N_PAD, D), jnp.float32),
            pltpu.SemaphoreType.DMA,
        ],
    )
    def agg_kernel(h_hbm, srcp_hbm, dstp_hbm, zeros_hbm, out_hbm,
                   src_v, dst_v, rows_v, acc_sh, sem):
        c = lax.axis_index("c")
        s = lax.axis_index("s")
        t = c * NS + s
        pltpu.sync_copy(zeros_hbm.at[pl.ds(s * ZROWS, ZROWS)],
                        acc_sh.at[pl.ds(s * ZROWS, ZROWS)])
        pltpu.sync_copy(srcp_hbm.at[t], src_v)
        pltpu.sync_copy(dstp_hbm.at[t], dst_v)
        plsc.subcore_barrier()

        def body(j, carry):
            pltpu.async_copy(h_hbm.at[src_v.at[j]], rows_v, sem).wait()
            pltpu.sync_copy(rows_v, acc_sh.at[dst_v.at[j]], add=True)
            return carry

        lax.fori_loop(0, NCH, body, 0)
        plsc.subcore_barrier()

        @pl.when(s < NS - 1)
        def _copy_full():
            pltpu.sync_copy(acc_sh.at[pl.ds(s * OROWS, OROWS)],
                            out_hbm.at[c, pl.ds(s * OROWS, OROWS)])

        @pl.when(s == NS - 1)
        def _copy_tail():
            pltpu.sync_copy(acc_sh.at[pl.ds((NS - 1) * OROWS, OTAIL)],
                            out_hbm.at[c, pl.ds((NS - 1) * OROWS, OTAIL)])

    return agg_kernel(h, srcp, dstp, zeros)


def _leaky(x):
    return jnp.where(x >= 0, x, 0.01 * x)


def _tc_mlp(h, agg, W1, b1, W2, b2, g, be, do_ln):
    def body(x_ref, a_ref, W1_ref, b1_ref, W2_ref, b2_ref, g_ref, be_ref, o_ref):
        t = x_ref[...] + a_ref[0] + a_ref[1]
        u = jnp.dot(t, W1_ref[...], preferred_element_type=jnp.float32) + b1_ref[...]
        u = _leaky(u)
        v = jnp.dot(u, W2_ref[...], preferred_element_type=jnp.float32) + b2_ref[...]
        if do_ln:
            v = _leaky(v)
            m = jnp.mean(v, axis=1, keepdims=True)
            var = jnp.mean((v - m) * (v - m), axis=1, keepdims=True)
            v = (v - m) * lax.rsqrt(var + 1e-5) * g_ref[...] + be_ref[...]
        o_ref[...] = v

    full = lambda i: (0, 0)
    return pl.pallas_call(
        body,
        grid=(GRID,),
        in_specs=[
            pl.BlockSpec((BN, D), lambda i: (i, 0)),
            pl.BlockSpec((NC, BN, D), lambda i: (0, i, 0)),
            pl.BlockSpec((D, D), full),
            pl.BlockSpec((1, D), full),
            pl.BlockSpec((D, D), full),
            pl.BlockSpec((1, D), full),
            pl.BlockSpec((1, D), full),
            pl.BlockSpec((1, D), full),
        ],
        out_specs=pl.BlockSpec((BN, D), lambda i: (i, 0)),
        out_shape=jax.ShapeDtypeStruct((N, D), jnp.float32),
    )(h, agg, W1, b1.reshape(1, D), W2, b2.reshape(1, D),
      g.reshape(1, D), be.reshape(1, D))


def _tc_mlp_pool(h, agg, W1, b1, W2, b2, batch2, Wp1, bp1, Wp2, bp2):
    def body(x_ref, a_ref, W1_ref, b1_ref, W2_ref, b2_ref, b_ref,
             Wp1_ref, bp1_ref, Wp2_ref, bp2_ref, e_ref, o_ref, acc_s, acc_c):
        i = pl.program_id(0)

        @pl.when(i == 0)
        def _init():
            acc_s[...] = jnp.zeros_like(acc_s)
            acc_c[...] = jnp.zeros_like(acc_c)

        t = x_ref[...] + a_ref[0] + a_ref[1]
        u = jnp.dot(t, W1_ref[...], preferred_element_type=jnp.float32) + b1_ref[...]
        u = _leaky(u)
        v = jnp.dot(u, W2_ref[...], preferred_element_type=jnp.float32) + b2_ref[...]
        e_ref[...] = v
        hp = _leaky(v)
        gids = b_ref[...]
        mask = (gids == lax.broadcasted_iota(jnp.int32, (1, G), 1)
                ).astype(jnp.float32)
        dn = (((0,), (0,)), ((), ()))
        acc_s[...] += lax.dot_general(mask, hp, dn,
                                      preferred_element_type=jnp.float32)
        acc_c[...] += lax.dot_general(mask, jnp.ones_like(hp), dn,
                                      preferred_element_type=jnp.float32)

        @pl.when(i == GRID - 1)
        def _fin():
            pooled = acc_s[...] / jnp.maximum(acc_c[...], 1.0)
            o = jnp.dot(pooled, Wp1_ref[...],
                        preferred_element_type=jnp.float32) + bp1_ref[...]
            o = jnp.dot(o, Wp2_ref[...],
                        preferred_element_type=jnp.float32) + bp2_ref[...]
            mx = jnp.max(o, axis=1, keepdims=True)
            lse = jnp.log(jnp.sum(jnp.exp(o - mx), axis=1, keepdims=True))
            o_ref[...] = o - mx - lse

    full = lambda i: (0, 0)
    return pl.pallas_call(
        body,
        grid=(GRID,),
        in_specs=[
            pl.BlockSpec((BN, D), lambda i: (i, 0)),
            pl.BlockSpec((NC, BN, D), lambda i: (0, i, 0)),
            pl.BlockSpec((D, D), full),
            pl.BlockSpec((1, D), full),
            pl.BlockSpec((D, D), full),
            pl.BlockSpec((1, D), full),
            pl.BlockSpec((BN, 1), lambda i: (i, 0)),
            pl.BlockSpec((D, D), full),
            pl.BlockSpec((1, D), full),
            pl.BlockSpec((D, D), full),
            pl.BlockSpec((1, D), full),
        ],
        out_specs=[
            pl.BlockSpec((BN, D), lambda i: (i, 0)),
            pl.BlockSpec((G, D), full),
        ],
        out_shape=[
            jax.ShapeDtypeStruct((N, D), jnp.float32),
            jax.ShapeDtypeStruct((G, D), jnp.float32),
        ],
        scratch_shapes=[
            pltpu.VMEM((G, D), jnp.float32),
            pltpu.VMEM((G, D), jnp.float32),
        ],
    )(h, agg, W1, b1.reshape(1, D), W2, b2.reshape(1, D), batch2,
      Wp1, bp1.reshape(1, D), Wp2, bp2.reshape(1, D))


def kernel(x, edge_index, batch,
           W1_0, b1_0, W2_0, b2_0,
           W1_1, b1_1, W2_1, b2_1,
           W1_2, b1_2, W2_2, b2_2,
           g0, be0, g1, be1,
           Wp1, bp1, Wp2, bp2):
    src = edge_index[0]
    dst = edge_index[1]
    pad = EPT_PAD - EPT
    srcp = jnp.pad(src.reshape(NW, EPT), ((0, 0), (0, pad))
                   ).reshape(NW, NCH, CH)
    dummy = (N + jnp.arange(NW, dtype=jnp.int32) % (N_PAD - N))[:, None]
    dstp = jnp.concatenate(
        [dst.reshape(NW, EPT), jnp.broadcast_to(dummy, (NW, pad))],
        axis=1).reshape(NW, NCH, CH)
    zeros = jnp.zeros((N_PAD, D), jnp.float32)
    batch2 = batch.reshape(N, 1)

    convs = [(W1_0, b1_0, W2_0, b2_0), (W1_1, b1_1, W2_1, b2_1)]
    lns = [(g0, be0), (g1, be1)]

    h = x
    for i in range(2):
        agg = _sc_aggregate(h, srcp, dstp, zeros)
        W1, b1, W2, b2 = convs[i]
        g, be = lns[i]
        h = _tc_mlp(h, agg, W1, b1, W2, b2, g, be, do_ln=True)
    agg = _sc_aggregate(h, srcp, dstp, zeros)
    emb, logp = _tc_mlp_pool(h, agg, W1_2, b1_2, W2_2, b2_2, batch2,
                             Wp1, bp1, Wp2, bp2)
    return (emb, logp)
